# baseline (speedup 1.0000x reference)
import functools

import jax
import jax.numpy as jnp
from jax import lax
from jax.experimental import pallas as pl
from jax.experimental.pallas import tpu as pltpu
from jax.experimental.pallas import tpu_sc as plsc

N = 10000
NPAD = 10112
E = 320000
NW = 32
NB = 80
BATCH = 128
EPAD = NW * NB * BATCH
RPT = NPAD // 16

_mesh = plsc.VectorSubcoreMesh(core_axis_name="c", subcore_axis_name="s")


NBUF = 2
NGR = NB // NBUF


def _sc_layer1_body(p_hbm, src_hbm, dst_hbm, z32_hbm, z16_hbm, ones_hbm,
                    sum_out, cnt_out, idx_s, idx_d, rows, ones_v,
                    acc, cnt, *semlist):
    semg, sems, semo = semlist[:NBUF], semlist[NBUF:2 * NBUF], semlist[-1]
    cid = lax.axis_index("c")
    sid = lax.axis_index("s")
    wid = cid * 16 + sid
    r0 = sid * RPT
    pltpu.sync_copy(z32_hbm, acc.at[pl.ds(r0, RPT)])
    pltpu.sync_copy(z16_hbm, cnt.at[pl.ds(r0, RPT)])
    pltpu.sync_copy(ones_hbm, ones_v)
    pltpu.sync_copy(src_hbm.at[wid], idx_s)
    pltpu.sync_copy(dst_hbm.at[wid], idx_d)
    plsc.subcore_barrier()

    for b in range(NBUF):
        pltpu.async_copy(p_hbm.at[idx_s.at[b]], rows.at[b], semg[b])

    def body(g, carry):
        base = g * NBUF
        for b in range(NBUF):
            pltpu.make_async_copy(p_hbm.at[idx_s.at[0]],
                                  rows.at[b], semg[b]).wait()
            pltpu.async_copy(rows.at[b], acc.at[idx_d.at[base + b]],
                             sems[b], add=True)
            pltpu.async_copy(ones_v, cnt.at[idx_d.at[base + b]],
                             semo, add=True)

        @pl.when(g + 1 < NGR)
        def _():
            for b in range(NBUF):
                pltpu.make_async_copy(rows.at[b], acc.at[idx_d.at[0]],
                                      sems[b]).wait()
                pltpu.async_copy(p_hbm.at[idx_s.at[base + NBUF + b]],
                                 rows.at[b], semg[b])
        return carry

    lax.fori_loop(0, NGR, body, 0)
    for b in range(NBUF):
        pltpu.make_async_copy(rows.at[b], acc.at[idx_d.at[0]], sems[b]).wait()

    def drain(j, carry):
        pltpu.make_async_copy(ones_v, cnt.at[idx_d.at[0]], semo).wait()
        return carry

    lax.fori_loop(0, NB, drain, 0)
    plsc.subcore_barrier()
    pltpu.sync_copy(acc.at[pl.ds(r0, RPT)], sum_out.at[cid, pl.ds(r0, RPT)])
    pltpu.sync_copy(cnt.at[pl.ds(r0, RPT)], cnt_out.at[cid, pl.ds(r0, RPT)])


def _sc_layer2_body(p_hbm, src_hbm, dst_hbm, z16_hbm,
                    sum_out, idx_s, idx_d, rows, acc, *semlist):
    semg, sems = semlist[:NBUF], semlist[NBUF:2 * NBUF]
    cid = lax.axis_index("c")
    sid = lax.axis_index("s")
    wid = cid * 16 + sid
    r0 = sid * RPT
    pltpu.sync_copy(z16_hbm, acc.at[pl.ds(r0, RPT)])
    pltpu.sync_copy(src_hbm.at[wid], idx_s)
    pltpu.sync_copy(dst_hbm.at[wid], idx_d)
    plsc.subcore_barrier()

    for b in range(NBUF):
        pltpu.async_copy(p_hbm.at[idx_s.at[b]], rows.at[b], semg[b])

    def body(g, carry):
        base = g * NBUF
        for b in range(NBUF):
            pltpu.make_async_copy(p_hbm.at[idx_s.at[0]],
                                  rows.at[b], semg[b]).wait()
            pltpu.async_copy(rows.at[b], acc.at[idx_d.at[base + b]],
                             sems[b], add=True)

        @pl.when(g + 1 < NGR)
        def _():
            for b in range(NBUF):
                pltpu.make_async_copy(rows.at[b], acc.at[idx_d.at[0]],
                                      sems[b]).wait()
                pltpu.async_copy(p_hbm.at[idx_s.at[base + NBUF + b]],
                                 rows.at[b], semg[b])
        return carry

    lax.fori_loop(0, NGR, body, 0)
    for b in range(NBUF):
        pltpu.make_async_copy(rows.at[b], acc.at[idx_d.at[0]], sems[b]).wait()
    plsc.subcore_barrier()
    pltpu.sync_copy(acc.at[pl.ds(r0, RPT)], sum_out.at[cid, pl.ds(r0, RPT)])


_sc_layer1 = pl.kernel(
    _sc_layer1_body,
    out_type=[jax.ShapeDtypeStruct((2, NPAD, 32), jnp.float32),
              jax.ShapeDtypeStruct((2, NPAD, 16), jnp.float32)],
    mesh=_mesh,
    compiler_params=pltpu.CompilerParams(use_tc_tiling_on_sc=False),
    scratch_types=(
        [pltpu.VMEM((NB, BATCH), jnp.int32),
         pltpu.VMEM((NB, BATCH), jnp.int32),
         pltpu.VMEM((NBUF, BATCH, 32), jnp.float32),
         pltpu.VMEM((BATCH, 16), jnp.float32),
         pltpu.VMEM_SHARED((NPAD, 32), jnp.float32),
         pltpu.VMEM_SHARED((NPAD, 16), jnp.float32)]
        + [pltpu.SemaphoreType.DMA] * (2 * NBUF + 1)))

_sc_layer2 = pl.kernel(
    _sc_layer2_body,
    out_type=[jax.ShapeDtypeStruct((2, NPAD, 16), jnp.float32)],
    mesh=_mesh,
    compiler_params=pltpu.CompilerParams(use_tc_tiling_on_sc=False),
    scratch_types=(
        [pltpu.VMEM((NB, BATCH), jnp.int32),
         pltpu.VMEM((NB, BATCH), jnp.int32),
         pltpu.VMEM((NBUF, BATCH, 16), jnp.float32),
         pltpu.VMEM_SHARED((NPAD, 16), jnp.float32)]
        + [pltpu.SemaphoreType.DMA] * (2 * NBUF)))


def _tc_in_body(x_ref, wl_ref, wr_ref, bl_ref, p_ref, q_ref):
    x = x_ref[...]
    p_ref[...] = jnp.dot(x, wl_ref[...], preferred_element_type=jnp.float32)
    q_ref[...] = (jnp.dot(x, wr_ref[...], preferred_element_type=jnp.float32)
                  + bl_ref[...])


def _tc_mid_body(sum_ref, cnt_ref, q1_ref, wl_ref, wr_ref, bl_ref,
                 p2_ref, q2_ref):
    s = sum_ref[0, :N, :] + sum_ref[1, :N, :]
    c = cnt_ref[0, :N, :1] + cnt_ref[1, :N, :1]
    h = jnp.maximum(s / jnp.maximum(c, 1.0) + q1_ref[...], 0.0)
    p2_ref[...] = jnp.dot(h, wl_ref[...], preferred_element_type=jnp.float32)
    q2_ref[...] = (jnp.dot(h, wr_ref[...], preferred_element_type=jnp.float32)
                   + bl_ref[...])


def _tc_out_body(sum_ref, cnt_ref, q2_ref, wo_ref, bo_ref, o_ref):
    s = sum_ref[0, :N, :] + sum_ref[1, :N, :]
    c = cnt_ref[0, :N, :1] + cnt_ref[1, :N, :1]
    h = jnp.maximum(s / jnp.maximum(c, 1.0) + q2_ref[...], 0.0)
    o_ref[...] = jax.nn.sigmoid(
        jnp.dot(h, wo_ref[...], preferred_element_type=jnp.float32)
        + bo_ref[...])


_tc_in = pl.pallas_call(
    _tc_in_body,
    out_shape=[jax.ShapeDtypeStruct((N, 32), jnp.float32),
               jax.ShapeDtypeStruct((N, 32), jnp.float32)])
_tc_mid = pl.pallas_call(
    _tc_mid_body,
    out_shape=[jax.ShapeDtypeStruct((N, 16), jnp.float32),
               jax.ShapeDtypeStruct((N, 16), jnp.float32)])
_tc_out = pl.pallas_call(
    _tc_out_body,
    out_shape=jax.ShapeDtypeStruct((N, 8), jnp.float32))


def kernel(x, edge_index, Wl1, bl1, Wr1, Wl2, bl2, Wr2, Wo, bo):
    ei = edge_index.astype(jnp.int32)
    src = jnp.concatenate(
        [ei[0], jnp.zeros((EPAD - E,), jnp.int32)]).reshape(NW, NB, BATCH)
    dst = jnp.concatenate(
        [ei[1], jnp.full((EPAD - E,), N, jnp.int32)]).reshape(NW, NB, BATCH)
    z32 = jnp.zeros((RPT, 32), jnp.float32)
    z16 = jnp.zeros((RPT, 16), jnp.float32)
    ones = jnp.ones((BATCH, 16), jnp.float32)

    p1, q1 = _tc_in(x, Wl1, Wr1, bl1.reshape(1, 32))
    sum1, cnt = _sc_layer1(p1, src, dst, z32, z16, ones)
    p2, q2 = _tc_mid(sum1, cnt, q1, Wl2, Wr2, bl2.reshape(1, 16))
    sum2, = _sc_layer2(p2, src, dst, z16)
    o = _tc_out(sum2, cnt, q2, jnp.tile(Wo, (1, 8)), bo.reshape(1, 1))
    return o[:, :1]


# speedup vs baseline: 1.0704x; 1.0704x over previous
import functools

import jax
import jax.numpy as jnp
from jax import lax
from jax.experimental import pallas as pl
from jax.experimental.pallas import tpu as pltpu
from jax.experimental.pallas import tpu_sc as plsc

N = 10000
NPAD = 10112
E = 320000
NW = 32
NB = 80
BATCH = 128
EPAD = NW * NB * BATCH
RPT = NPAD // 16

_mesh = plsc.VectorSubcoreMesh(core_axis_name="c", subcore_axis_name="s")


CHUNK = 4
EC = CHUNK * BATCH
NC = NB // CHUNK
NCG = NC // 2


def _sc_layer1_body(p_hbm, src_hbm, dst_hbm, z32_hbm, z16_hbm, ones_hbm,
                    sum_out, cnt_out, idx_s, idx_d, rows, ones_v,
                    acc, cnt, semg0, semg1, semo):
    cid = lax.axis_index("c")
    sid = lax.axis_index("s")
    wid = cid * 16 + sid
    r0 = sid * RPT
    pltpu.sync_copy(z32_hbm, acc.at[pl.ds(r0, RPT)])
    pltpu.sync_copy(z16_hbm, cnt.at[pl.ds(r0, RPT)])
    pltpu.sync_copy(ones_hbm, ones_v)
    pltpu.sync_copy(src_hbm.at[wid], idx_s)
    pltpu.sync_copy(dst_hbm.at[wid], idx_d)
    plsc.subcore_barrier()

    semg = (semg0, semg1)

    def gfire(c, b):
        pltpu.async_copy(p_hbm.at[idx_s.at[c]],
                         rows.at[b], semg[b])

    def gwait(b):
        pltpu.make_async_copy(p_hbm.at[idx_s.at[0]],
                              rows.at[b], semg[b]).wait()

    def scat(c, b):
        d = idx_d.at[c]
        pltpu.sync_copy(rows.at[b], acc.at[d], add=True)
        pltpu.async_copy(ones_v, cnt.at[d], semo, add=True)

    gfire(0, 0)

    def body(g, carry):
        c0 = g * 2
        gfire(c0 + 1, 1)
        gwait(0)
        scat(c0, 0)

        @pl.when(c0 + 2 < NC)
        def _():
            gfire(c0 + 2, 0)

        gwait(1)
        scat(c0 + 1, 1)
        return carry

    lax.fori_loop(0, NCG, body, 0)

    def drain(j, carry):
        pltpu.make_async_copy(ones_v, cnt.at[idx_d.at[0]],
                              semo).wait()
        return carry

    lax.fori_loop(0, NC, drain, 0)
    plsc.subcore_barrier()
    pltpu.sync_copy(acc.at[pl.ds(r0, RPT)], sum_out.at[cid, pl.ds(r0, RPT)])
    pltpu.sync_copy(cnt.at[pl.ds(r0, RPT)], cnt_out.at[cid, pl.ds(r0, RPT)])


def _sc_layer2_body(p_hbm, src_hbm, dst_hbm, z16_hbm,
                    sum_out, idx_s, idx_d, rows, acc, semg0, semg1):
    cid = lax.axis_index("c")
    sid = lax.axis_index("s")
    wid = cid * 16 + sid
    r0 = sid * RPT
    pltpu.sync_copy(z16_hbm, acc.at[pl.ds(r0, RPT)])
    pltpu.sync_copy(src_hbm.at[wid], idx_s)
    pltpu.sync_copy(dst_hbm.at[wid], idx_d)
    plsc.subcore_barrier()

    semg = (semg0, semg1)

    def gfire(c, b):
        pltpu.async_copy(p_hbm.at[idx_s.at[c]],
                         rows.at[b], semg[b])

    def gwait(b):
        pltpu.make_async_copy(p_hbm.at[idx_s.at[0]],
                              rows.at[b], semg[b]).wait()

    def scat(c, b):
        pltpu.sync_copy(rows.at[b], acc.at[idx_d.at[c]],
                        add=True)

    gfire(0, 0)

    def body(g, carry):
        c0 = g * 2
        gfire(c0 + 1, 1)
        gwait(0)
        scat(c0, 0)

        @pl.when(c0 + 2 < NC)
        def _():
            gfire(c0 + 2, 0)

        gwait(1)
        scat(c0 + 1, 1)
        return carry

    lax.fori_loop(0, NCG, body, 0)
    plsc.subcore_barrier()
    pltpu.sync_copy(acc.at[pl.ds(r0, RPT)], sum_out.at[cid, pl.ds(r0, RPT)])


_sc_layer1 = pl.kernel(
    _sc_layer1_body,
    out_type=[jax.ShapeDtypeStruct((2, NPAD, 32), jnp.float32),
              jax.ShapeDtypeStruct((2, NPAD, 16), jnp.float32)],
    mesh=_mesh,
    compiler_params=pltpu.CompilerParams(use_tc_tiling_on_sc=False),
    scratch_types=(
        [pltpu.VMEM((NC, EC), jnp.int32),
         pltpu.VMEM((NC, EC), jnp.int32),
         pltpu.VMEM((2, EC, 32), jnp.float32),
         pltpu.VMEM((EC, 16), jnp.float32),
         pltpu.VMEM_SHARED((NPAD, 32), jnp.float32),
         pltpu.VMEM_SHARED((NPAD, 16), jnp.float32)]
        + [pltpu.SemaphoreType.DMA] * 3))

_sc_layer2 = pl.kernel(
    _sc_layer2_body,
    out_type=[jax.ShapeDtypeStruct((2, NPAD, 16), jnp.float32)],
    mesh=_mesh,
    compiler_params=pltpu.CompilerParams(use_tc_tiling_on_sc=False),
    scratch_types=(
        [pltpu.VMEM((NC, EC), jnp.int32),
         pltpu.VMEM((NC, EC), jnp.int32),
         pltpu.VMEM((2, EC, 16), jnp.float32),
         pltpu.VMEM_SHARED((NPAD, 16), jnp.float32)]
        + [pltpu.SemaphoreType.DMA] * 2))


def _tc_in_body(x_ref, wl_ref, wr_ref, bl_ref, p_ref, q_ref):
    x = x_ref[...]
    p_ref[...] = jnp.dot(x, wl_ref[...], preferred_element_type=jnp.float32)
    q_ref[...] = (jnp.dot(x, wr_ref[...], preferred_element_type=jnp.float32)
                  + bl_ref[...])


def _tc_mid_body(sum_ref, cnt_ref, q1_ref, wl_ref, wr_ref, bl_ref,
                 p2_ref, q2_ref):
    s = sum_ref[0, :N, :] + sum_ref[1, :N, :]
    c = cnt_ref[0, :N, :1] + cnt_ref[1, :N, :1]
    h = jnp.maximum(s / jnp.maximum(c, 1.0) + q1_ref[...], 0.0)
    p2_ref[...] = jnp.dot(h, wl_ref[...], preferred_element_type=jnp.float32)
    q2_ref[...] = (jnp.dot(h, wr_ref[...], preferred_element_type=jnp.float32)
                   + bl_ref[...])


def _tc_out_body(sum_ref, cnt_ref, q2_ref, wo_ref, bo_ref, o_ref):
    s = sum_ref[0, :N, :] + sum_ref[1, :N, :]
    c = cnt_ref[0, :N, :1] + cnt_ref[1, :N, :1]
    h = jnp.maximum(s / jnp.maximum(c, 1.0) + q2_ref[...], 0.0)
    o_ref[...] = jax.nn.sigmoid(
        jnp.dot(h, wo_ref[...], preferred_element_type=jnp.float32)
        + bo_ref[...])


_tc_in = pl.pallas_call(
    _tc_in_body,
    out_shape=[jax.ShapeDtypeStruct((N, 32), jnp.float32),
               jax.ShapeDtypeStruct((N, 32), jnp.float32)])
_tc_mid = pl.pallas_call(
    _tc_mid_body,
    out_shape=[jax.ShapeDtypeStruct((N, 16), jnp.float32),
               jax.ShapeDtypeStruct((N, 16), jnp.float32)])
_tc_out = pl.pallas_call(
    _tc_out_body,
    out_shape=jax.ShapeDtypeStruct((N, 8), jnp.float32))


def kernel(x, edge_index, Wl1, bl1, Wr1, Wl2, bl2, Wr2, Wo, bo):
    ei = edge_index.astype(jnp.int32)
    src = jnp.concatenate(
        [ei[0], jnp.zeros((EPAD - E,), jnp.int32)]).reshape(NW, NC, EC)
    dst = jnp.concatenate(
        [ei[1], jnp.full((EPAD - E,), N, jnp.int32)]).reshape(NW, NC, EC)
    z32 = jnp.zeros((RPT, 32), jnp.float32)
    z16 = jnp.zeros((RPT, 16), jnp.float32)
    ones = jnp.ones((EC, 16), jnp.float32)

    p1, q1 = _tc_in(x, Wl1, Wr1, bl1.reshape(1, 32))
    sum1, cnt = _sc_layer1(p1, src, dst, z32, z16, ones)
    p2, q2 = _tc_mid(sum1, cnt, q1, Wl2, Wr2, bl2.reshape(1, 16))
    sum2, = _sc_layer2(p2, src, dst, z16)
    o = _tc_out(sum2, cnt, q2, jnp.tile(Wo, (1, 8)), bo.reshape(1, 1))
    return o[:, :1]


# speedup vs baseline: 1.0998x; 1.0275x over previous
import functools

import jax
import jax.numpy as jnp
from jax import lax
from jax.experimental import pallas as pl
from jax.experimental.pallas import tpu as pltpu
from jax.experimental.pallas import tpu_sc as plsc

N = 10000
NPAD = 10112
E = 320000
NW = 32
NB = 80
BATCH = 128
EPAD = NW * NB * BATCH
RPT = NPAD // 16

_mesh = plsc.VectorSubcoreMesh(core_axis_name="c", subcore_axis_name="s")


CHUNK = 4
EC = CHUNK * BATCH
NC = NB // CHUNK
NCG = NC // 2
HR = 640


def _sc_layer1_body(p_hbm, src_hbm, dst_hbm, z32_hbm, z640_hbm, iota_hbm,
                    sum_out, cnt_out, idx_s, idx_d, rows, hcnt, iota_v,
                    acc, cnt, semg0, semg1):
    cid = lax.axis_index("c")
    sid = lax.axis_index("s")
    wid = cid * 16 + sid
    r0 = sid * RPT
    pltpu.sync_copy(z32_hbm, acc.at[pl.ds(r0, RPT)])
    pltpu.sync_copy(z640_hbm, hcnt)
    pltpu.sync_copy(iota_hbm, iota_v)
    pltpu.sync_copy(src_hbm.at[wid], idx_s)
    pltpu.sync_copy(dst_hbm.at[wid], idx_d)

    @pl.when(sid < 8)
    def _():
        pltpu.sync_copy(z640_hbm.at[pl.ds(0, HR // 8)],
                        cnt.at[pl.ds(sid * (HR // 8), HR // 8)])

    plsc.subcore_barrier()

    semg = (semg0, semg1)
    ones16 = jnp.ones((16,), jnp.float32)

    def gfire(c, b):
        pltpu.async_copy(p_hbm.at[idx_s.at[c]],
                         rows.at[b], semg[b])

    def gwait(b):
        pltpu.make_async_copy(p_hbm.at[idx_s.at[0]],
                              rows.at[b], semg[b]).wait()

    def scat(c, b):
        pltpu.sync_copy(rows.at[b], acc.at[idx_d.at[c]], add=True)

    def hist(c):
        def hbody(k, carry):
            d = idx_d[c, pl.ds(k * 16, 16)]
            plsc.addupdate_scatter(
                hcnt, [lax.shift_right_logical(d, 4),
                       jnp.bitwise_and(d, 15)], ones16)
            return carry
        lax.fori_loop(0, EC // 16, hbody, 0)

    gfire(0, 0)

    def body(g, carry):
        c0 = g * 2
        gfire(c0 + 1, 1)
        hist(c0)
        gwait(0)
        scat(c0, 0)

        @pl.when(c0 + 2 < NC)
        def _():
            gfire(c0 + 2, 0)

        hist(c0 + 1)
        gwait(1)
        scat(c0 + 1, 1)
        return carry

    lax.fori_loop(0, NCG, body, 0)
    pltpu.sync_copy(hcnt, cnt.at[iota_v], add=True)
    plsc.subcore_barrier()
    pltpu.sync_copy(acc.at[pl.ds(r0, RPT)], sum_out.at[cid, pl.ds(r0, RPT)])

    @pl.when(sid < 8)
    def _():
        pltpu.sync_copy(cnt.at[pl.ds(sid * (HR // 8), HR // 8)],
                        cnt_out.at[cid, pl.ds(sid * (HR // 8), HR // 8)])


def _sc_layer2_body(p_hbm, src_hbm, dst_hbm, z16_hbm,
                    sum_out, idx_s, idx_d, rows, acc, semg0, semg1):
    cid = lax.axis_index("c")
    sid = lax.axis_index("s")
    wid = cid * 16 + sid
    r0 = sid * RPT
    pltpu.sync_copy(z16_hbm, acc.at[pl.ds(r0, RPT)])
    pltpu.sync_copy(src_hbm.at[wid], idx_s)
    pltpu.sync_copy(dst_hbm.at[wid], idx_d)
    plsc.subcore_barrier()

    semg = (semg0, semg1)

    def gfire(c, b):
        pltpu.async_copy(p_hbm.at[idx_s.at[c]],
                         rows.at[b], semg[b])

    def gwait(b):
        pltpu.make_async_copy(p_hbm.at[idx_s.at[0]],
                              rows.at[b], semg[b]).wait()

    def scat(c, b):
        pltpu.sync_copy(rows.at[b], acc.at[idx_d.at[c]],
                        add=True)

    gfire(0, 0)

    def body(g, carry):
        c0 = g * 2
        gfire(c0 + 1, 1)
        gwait(0)
        scat(c0, 0)

        @pl.when(c0 + 2 < NC)
        def _():
            gfire(c0 + 2, 0)

        gwait(1)
        scat(c0 + 1, 1)
        return carry

    lax.fori_loop(0, NCG, body, 0)
    plsc.subcore_barrier()
    pltpu.sync_copy(acc.at[pl.ds(r0, RPT)], sum_out.at[cid, pl.ds(r0, RPT)])


_sc_layer1 = pl.kernel(
    _sc_layer1_body,
    out_type=[jax.ShapeDtypeStruct((2, NPAD, 32), jnp.float32),
              jax.ShapeDtypeStruct((2, HR, 16), jnp.float32)],
    mesh=_mesh,
    compiler_params=pltpu.CompilerParams(use_tc_tiling_on_sc=False,
                                         needs_layout_passes=False),
    scratch_types=(
        [pltpu.VMEM((NC, EC), jnp.int32),
         pltpu.VMEM((NC, EC), jnp.int32),
         pltpu.VMEM((2, EC, 32), jnp.float32),
         pltpu.VMEM((HR, 16), jnp.float32),
         pltpu.VMEM((HR,), jnp.int32),
         pltpu.VMEM_SHARED((NPAD, 32), jnp.float32),
         pltpu.VMEM_SHARED((HR, 16), jnp.float32)]
        + [pltpu.SemaphoreType.DMA] * 2))

_sc_layer2 = pl.kernel(
    _sc_layer2_body,
    out_type=[jax.ShapeDtypeStruct((2, NPAD, 16), jnp.float32)],
    mesh=_mesh,
    compiler_params=pltpu.CompilerParams(use_tc_tiling_on_sc=False),
    scratch_types=(
        [pltpu.VMEM((NC, EC), jnp.int32),
         pltpu.VMEM((NC, EC), jnp.int32),
         pltpu.VMEM((2, EC, 16), jnp.float32),
         pltpu.VMEM_SHARED((NPAD, 16), jnp.float32)]
        + [pltpu.SemaphoreType.DMA] * 2))


def _tc_in_body(x_ref, wl_ref, wr_ref, bl_ref, p_ref, q_ref):
    x = x_ref[...]
    p_ref[...] = jnp.dot(x, wl_ref[...], preferred_element_type=jnp.float32)
    q_ref[...] = (jnp.dot(x, wr_ref[...], preferred_element_type=jnp.float32)
                  + bl_ref[...])


def _tc_mid_body(sum_ref, cnt_ref, q1_ref, wl_ref, wr_ref, bl_ref,
                 p2_ref, q2_ref):
    s = sum_ref[0, :N, :] + sum_ref[1, :N, :]
    c = cnt_ref[0, :N, :1] + cnt_ref[1, :N, :1]
    h = jnp.maximum(s / jnp.maximum(c, 1.0) + q1_ref[...], 0.0)
    p2_ref[...] = jnp.dot(h, wl_ref[...], preferred_element_type=jnp.float32)
    q2_ref[...] = (jnp.dot(h, wr_ref[...], preferred_element_type=jnp.float32)
                   + bl_ref[...])


def _tc_out_body(sum_ref, cnt_ref, q2_ref, wo_ref, bo_ref, o_ref):
    s = sum_ref[0, :N, :] + sum_ref[1, :N, :]
    c = cnt_ref[0, :N, :1] + cnt_ref[1, :N, :1]
    h = jnp.maximum(s / jnp.maximum(c, 1.0) + q2_ref[...], 0.0)
    o_ref[...] = jax.nn.sigmoid(
        jnp.dot(h, wo_ref[...], preferred_element_type=jnp.float32)
        + bo_ref[...])


_tc_in = pl.pallas_call(
    _tc_in_body,
    out_shape=[jax.ShapeDtypeStruct((N, 32), jnp.float32),
               jax.ShapeDtypeStruct((N, 32), jnp.float32)])
_tc_mid = pl.pallas_call(
    _tc_mid_body,
    out_shape=[jax.ShapeDtypeStruct((N, 16), jnp.float32),
               jax.ShapeDtypeStruct((N, 16), jnp.float32)])
_tc_out = pl.pallas_call(
    _tc_out_body,
    out_shape=jax.ShapeDtypeStruct((N, 8), jnp.float32))


def kernel(x, edge_index, Wl1, bl1, Wr1, Wl2, bl2, Wr2, Wo, bo):
    ei = edge_index.astype(jnp.int32)
    src = jnp.concatenate(
        [ei[0], jnp.zeros((EPAD - E,), jnp.int32)]).reshape(NW, NC, EC)
    dst = jnp.concatenate(
        [ei[1], jnp.full((EPAD - E,), N, jnp.int32)]).reshape(NW, NC, EC)
    z32 = jnp.zeros((RPT, 32), jnp.float32)
    z16 = jnp.zeros((RPT, 16), jnp.float32)
    z640 = jnp.zeros((HR, 16), jnp.float32)
    iota = jnp.arange(HR, dtype=jnp.int32)

    p1, q1 = _tc_in(x, Wl1, Wr1, bl1.reshape(1, 32))
    sum1, cnt = _sc_layer1(p1, src, dst, z32, z640, iota)
    cntp = cnt.reshape(2, HR * 16, 1)
    p2, q2 = _tc_mid(sum1, cntp, q1, Wl2, Wr2, bl2.reshape(1, 16))
    sum2, = _sc_layer2(p2, src, dst, z16)
    o = _tc_out(sum2, cntp, q2, jnp.tile(Wo, (1, 8)), bo.reshape(1, 1))
    return o[:, :1]


# speedup vs baseline: 1.7173x; 1.5615x over previous
import functools

import jax
import jax.numpy as jnp
from jax import lax
from jax.experimental import pallas as pl
from jax.experimental.pallas import tpu as pltpu
from jax.experimental.pallas import tpu_sc as plsc

N = 10000
NPAD = 10112
E = 320000
NW = 32
NB = 80
BATCH = 128
EPAD = NW * NB * BATCH
RPT = NPAD // 16

_mesh = plsc.VectorSubcoreMesh(core_axis_name="c", subcore_axis_name="s")


CHUNK = 4
EC = CHUNK * BATCH
NC = NB // CHUNK
NCG = NC // 2
HR = 640
SRT = N // 16


def _sc_layer1_body(p_hbm, src_hbm, dst_hbm, z32_hbm, z640_hbm, iota_hbm,
                    sum_out, cnt_out, idx_s, idx_d, rows, hcnt, iota_v,
                    p_sh, acc, cnt, semg0, semg1):
    cid = lax.axis_index("c")
    sid = lax.axis_index("s")
    wid = cid * 16 + sid
    r0 = sid * RPT
    pltpu.sync_copy(z32_hbm, acc.at[pl.ds(r0, RPT)])
    pltpu.sync_copy(z640_hbm, hcnt)
    pltpu.sync_copy(iota_hbm, iota_v)
    pltpu.sync_copy(src_hbm.at[wid], idx_s)
    pltpu.sync_copy(dst_hbm.at[wid], idx_d)
    pltpu.sync_copy(p_hbm.at[pl.ds(sid * SRT, SRT)],
                    p_sh.at[pl.ds(sid * SRT, SRT)])

    @pl.when(sid < 8)
    def _():
        pltpu.sync_copy(z640_hbm.at[pl.ds(0, HR // 8)],
                        cnt.at[pl.ds(sid * (HR // 8), HR // 8)])

    plsc.subcore_barrier()

    semg = (semg0, semg1)
    ones16 = jnp.ones((16,), jnp.float32)

    def gfire(c, b):
        pltpu.async_copy(p_sh.at[idx_s.at[c]],
                         rows.at[b], semg[b])

    def gwait(b):
        pltpu.make_async_copy(p_sh.at[idx_s.at[0]],
                              rows.at[b], semg[b]).wait()

    def scat(c, b):
        pltpu.sync_copy(rows.at[b], acc.at[idx_d.at[c]], add=True)

    def hist(c):
        def hbody(k, carry):
            d = idx_d[c, pl.ds(k * 16, 16)]
            plsc.addupdate_scatter(
                hcnt, [lax.shift_right_logical(d, 4),
                       jnp.bitwise_and(d, 15)], ones16)
            return carry
        lax.fori_loop(0, EC // 16, hbody, 0)

    gfire(0, 0)

    def body(g, carry):
        c0 = g * 2
        gfire(c0 + 1, 1)
        hist(c0)
        gwait(0)
        scat(c0, 0)

        @pl.when(c0 + 2 < NC)
        def _():
            gfire(c0 + 2, 0)

        hist(c0 + 1)
        gwait(1)
        scat(c0 + 1, 1)
        return carry

    lax.fori_loop(0, NCG, body, 0)
    pltpu.sync_copy(hcnt, cnt.at[iota_v], add=True)
    plsc.subcore_barrier()
    pltpu.sync_copy(acc.at[pl.ds(r0, RPT)], sum_out.at[cid, pl.ds(r0, RPT)])

    @pl.when(sid < 8)
    def _():
        pltpu.sync_copy(cnt.at[pl.ds(sid * (HR // 8), HR // 8)],
                        cnt_out.at[cid, pl.ds(sid * (HR // 8), HR // 8)])


def _sc_layer2_body(p_hbm, src_hbm, dst_hbm, z16_hbm,
                    sum_out, idx_s, idx_d, rows, p_sh, acc, semg0, semg1):
    cid = lax.axis_index("c")
    sid = lax.axis_index("s")
    wid = cid * 16 + sid
    r0 = sid * RPT
    pltpu.sync_copy(z16_hbm, acc.at[pl.ds(r0, RPT)])
    pltpu.sync_copy(src_hbm.at[wid], idx_s)
    pltpu.sync_copy(dst_hbm.at[wid], idx_d)
    pltpu.sync_copy(p_hbm.at[pl.ds(sid * SRT, SRT)],
                    p_sh.at[pl.ds(sid * SRT, SRT)])
    plsc.subcore_barrier()

    semg = (semg0, semg1)

    def gfire(c, b):
        pltpu.async_copy(p_sh.at[idx_s.at[c]],
                         rows.at[b], semg[b])

    def gwait(b):
        pltpu.make_async_copy(p_sh.at[idx_s.at[0]],
                              rows.at[b], semg[b]).wait()

    def scat(c, b):
        pltpu.sync_copy(rows.at[b], acc.at[idx_d.at[c]],
                        add=True)

    gfire(0, 0)

    def body(g, carry):
        c0 = g * 2
        gfire(c0 + 1, 1)
        gwait(0)
        scat(c0, 0)

        @pl.when(c0 + 2 < NC)
        def _():
            gfire(c0 + 2, 0)

        gwait(1)
        scat(c0 + 1, 1)
        return carry

    lax.fori_loop(0, NCG, body, 0)
    plsc.subcore_barrier()
    pltpu.sync_copy(acc.at[pl.ds(r0, RPT)], sum_out.at[cid, pl.ds(r0, RPT)])


_sc_layer1 = pl.kernel(
    _sc_layer1_body,
    out_type=[jax.ShapeDtypeStruct((2, NPAD, 32), jnp.float32),
              jax.ShapeDtypeStruct((2, HR, 16), jnp.float32)],
    mesh=_mesh,
    compiler_params=pltpu.CompilerParams(use_tc_tiling_on_sc=False,
                                         needs_layout_passes=False),
    scratch_types=(
        [pltpu.VMEM((NC, EC), jnp.int32),
         pltpu.VMEM((NC, EC), jnp.int32),
         pltpu.VMEM((2, EC, 32), jnp.float32),
         pltpu.VMEM((HR, 16), jnp.float32),
         pltpu.VMEM((HR,), jnp.int32),
         pltpu.VMEM_SHARED((N, 32), jnp.float32),
         pltpu.VMEM_SHARED((NPAD, 32), jnp.float32),
         pltpu.VMEM_SHARED((HR, 16), jnp.float32)]
        + [pltpu.SemaphoreType.DMA] * 2))

_sc_layer2 = pl.kernel(
    _sc_layer2_body,
    out_type=[jax.ShapeDtypeStruct((2, NPAD, 16), jnp.float32)],
    mesh=_mesh,
    compiler_params=pltpu.CompilerParams(use_tc_tiling_on_sc=False),
    scratch_types=(
        [pltpu.VMEM((NC, EC), jnp.int32),
         pltpu.VMEM((NC, EC), jnp.int32),
         pltpu.VMEM((2, EC, 16), jnp.float32),
         pltpu.VMEM_SHARED((N, 16), jnp.float32),
         pltpu.VMEM_SHARED((NPAD, 16), jnp.float32)]
        + [pltpu.SemaphoreType.DMA] * 2))


def _tc_in_body(x_ref, wl_ref, wr_ref, bl_ref, p_ref, q_ref):
    x = x_ref[...]
    p_ref[...] = jnp.dot(x, wl_ref[...], preferred_element_type=jnp.float32)
    q_ref[...] = (jnp.dot(x, wr_ref[...], preferred_element_type=jnp.float32)
                  + bl_ref[...])


def _tc_mid_body(sum_ref, cnt_ref, q1_ref, wl_ref, wr_ref, bl_ref,
                 p2_ref, q2_ref):
    s = sum_ref[0, :N, :] + sum_ref[1, :N, :]
    c = cnt_ref[0, :N, :1] + cnt_ref[1, :N, :1]
    h = jnp.maximum(s / jnp.maximum(c, 1.0) + q1_ref[...], 0.0)
    p2_ref[...] = jnp.dot(h, wl_ref[...], preferred_element_type=jnp.float32)
    q2_ref[...] = (jnp.dot(h, wr_ref[...], preferred_element_type=jnp.float32)
                   + bl_ref[...])


def _tc_out_body(sum_ref, cnt_ref, q2_ref, wo_ref, bo_ref, o_ref):
    s = sum_ref[0, :N, :] + sum_ref[1, :N, :]
    c = cnt_ref[0, :N, :1] + cnt_ref[1, :N, :1]
    h = jnp.maximum(s / jnp.maximum(c, 1.0) + q2_ref[...], 0.0)
    o_ref[...] = jax.nn.sigmoid(
        jnp.dot(h, wo_ref[...], preferred_element_type=jnp.float32)
        + bo_ref[...])


_tc_in = pl.pallas_call(
    _tc_in_body,
    out_shape=[jax.ShapeDtypeStruct((N, 32), jnp.float32),
               jax.ShapeDtypeStruct((N, 32), jnp.float32)])
_tc_mid = pl.pallas_call(
    _tc_mid_body,
    out_shape=[jax.ShapeDtypeStruct((N, 16), jnp.float32),
               jax.ShapeDtypeStruct((N, 16), jnp.float32)])
_tc_out = pl.pallas_call(
    _tc_out_body,
    out_shape=jax.ShapeDtypeStruct((N, 8), jnp.float32))


def kernel(x, edge_index, Wl1, bl1, Wr1, Wl2, bl2, Wr2, Wo, bo):
    ei = edge_index.astype(jnp.int32)
    src = jnp.concatenate(
        [ei[0], jnp.zeros((EPAD - E,), jnp.int32)]).reshape(NW, NC, EC)
    dst = jnp.concatenate(
        [ei[1], jnp.full((EPAD - E,), N, jnp.int32)]).reshape(NW, NC, EC)
    z32 = jnp.zeros((RPT, 32), jnp.float32)
    z16 = jnp.zeros((RPT, 16), jnp.float32)
    z640 = jnp.zeros((HR, 16), jnp.float32)
    iota = jnp.arange(HR, dtype=jnp.int32)

    p1, q1 = _tc_in(x, Wl1, Wr1, bl1.reshape(1, 32))
    sum1, cnt = _sc_layer1(p1, src, dst, z32, z640, iota)
    cntp = cnt.reshape(2, HR * 16, 1)
    p2, q2 = _tc_mid(sum1, cntp, q1, Wl2, Wr2, bl2.reshape(1, 16))
    sum2, = _sc_layer2(p2, src, dst, z16)
    o = _tc_out(sum2, cntp, q2, jnp.tile(Wo, (1, 8)), bo.reshape(1, 1))
    return o[:, :1]


# speedup vs baseline: 1.8436x; 1.0735x over previous
import functools

import jax
import jax.numpy as jnp
from jax import lax
from jax.experimental import pallas as pl
from jax.experimental.pallas import tpu as pltpu
from jax.experimental.pallas import tpu_sc as plsc

N = 10000
NPAD = 10112
E = 320000
NW = 32
RPT = NPAD // 16

_mesh = plsc.VectorSubcoreMesh(core_axis_name="c", subcore_axis_name="s")


EC = 400
NC = 25
NCG = NC // 2
HR = 640
SRT = N // 16


def _sc_layer1_body(p_hbm, src_hbm, dst_hbm, z32_hbm, z640_hbm, iota_hbm,
                    sum_out, cnt_out, idx_s, idx_d, rows, hcnt, iota_v,
                    p_sh, acc, cnt, semg0, semg1):
    cid = lax.axis_index("c")
    sid = lax.axis_index("s")
    wid = cid * 16 + sid
    r0 = sid * RPT
    pltpu.sync_copy(z32_hbm, acc.at[pl.ds(r0, RPT)])
    pltpu.sync_copy(z640_hbm, hcnt)
    pltpu.sync_copy(iota_hbm, iota_v)
    pltpu.sync_copy(src_hbm.at[wid], idx_s)
    pltpu.sync_copy(dst_hbm.at[wid], idx_d)
    pltpu.sync_copy(p_hbm.at[pl.ds(sid * SRT, SRT)],
                    p_sh.at[pl.ds(sid * SRT, SRT)])

    @pl.when(sid < 8)
    def _():
        pltpu.sync_copy(z640_hbm.at[pl.ds(0, HR // 8)],
                        cnt.at[pl.ds(sid * (HR // 8), HR // 8)])

    plsc.subcore_barrier()

    semg = (semg0, semg1)
    ones16 = jnp.ones((16,), jnp.float32)

    def gfire(c, b):
        pltpu.async_copy(p_sh.at[idx_s.at[c]],
                         rows.at[b], semg[b])

    def gwait(b):
        pltpu.make_async_copy(p_sh.at[idx_s.at[0]],
                              rows.at[b], semg[b]).wait()

    def scat(c, b):
        pltpu.sync_copy(rows.at[b], acc.at[idx_d.at[c]], add=True)

    def hist(c):
        def hbody(k, carry):
            d = idx_d[c, pl.ds(k * 16, 16)]
            plsc.addupdate_scatter(
                hcnt, [lax.shift_right_logical(d, 4),
                       jnp.bitwise_and(d, 15)], ones16)
            return carry
        lax.fori_loop(0, EC // 16, hbody, 0)

    gfire(0, 0)

    def body(g, carry):
        c0 = g * 2
        gfire(c0 + 1, 1)
        hist(c0)
        gwait(0)
        scat(c0, 0)

        @pl.when(c0 + 2 < NC)
        def _():
            gfire(c0 + 2, 0)

        hist(c0 + 1)
        gwait(1)
        scat(c0 + 1, 1)
        return carry

    lax.fori_loop(0, NCG, body, 0)
    hist(NC - 1)
    gwait(0)
    scat(NC - 1, 0)
    pltpu.sync_copy(hcnt, cnt.at[iota_v], add=True)
    plsc.subcore_barrier()
    pltpu.sync_copy(acc.at[pl.ds(r0, RPT)], sum_out.at[cid, pl.ds(r0, RPT)])

    @pl.when(sid < 8)
    def _():
        pltpu.sync_copy(cnt.at[pl.ds(sid * (HR // 8), HR // 8)],
                        cnt_out.at[cid, pl.ds(sid * (HR // 8), HR // 8)])


def _sc_layer2_body(p_hbm, src_hbm, dst_hbm, z16_hbm,
                    sum_out, idx_s, idx_d, rows, p_sh, acc, semg0, semg1):
    cid = lax.axis_index("c")
    sid = lax.axis_index("s")
    wid = cid * 16 + sid
    r0 = sid * RPT
    pltpu.sync_copy(z16_hbm, acc.at[pl.ds(r0, RPT)])
    pltpu.sync_copy(src_hbm.at[wid], idx_s)
    pltpu.sync_copy(dst_hbm.at[wid], idx_d)
    pltpu.sync_copy(p_hbm.at[pl.ds(sid * SRT, SRT)],
                    p_sh.at[pl.ds(sid * SRT, SRT)])
    plsc.subcore_barrier()

    semg = (semg0, semg1)

    def gfire(c, b):
        pltpu.async_copy(p_sh.at[idx_s.at[c]],
                         rows.at[b], semg[b])

    def gwait(b):
        pltpu.make_async_copy(p_sh.at[idx_s.at[0]],
                              rows.at[b], semg[b]).wait()

    def scat(c, b):
        pltpu.sync_copy(rows.at[b], acc.at[idx_d.at[c]],
                        add=True)

    gfire(0, 0)

    def body(g, carry):
        c0 = g * 2
        gfire(c0 + 1, 1)
        gwait(0)
        scat(c0, 0)

        @pl.when(c0 + 2 < NC)
        def _():
            gfire(c0 + 2, 0)

        gwait(1)
        scat(c0 + 1, 1)
        return carry

    lax.fori_loop(0, NCG, body, 0)
    gwait(0)
    scat(NC - 1, 0)
    plsc.subcore_barrier()
    pltpu.sync_copy(acc.at[pl.ds(r0, RPT)], sum_out.at[cid, pl.ds(r0, RPT)])


_sc_layer1 = pl.kernel(
    _sc_layer1_body,
    out_type=[jax.ShapeDtypeStruct((2, NPAD, 32), jnp.float32),
              jax.ShapeDtypeStruct((2, HR, 16), jnp.float32)],
    mesh=_mesh,
    compiler_params=pltpu.CompilerParams(use_tc_tiling_on_sc=False,
                                         needs_layout_passes=False),
    scratch_types=(
        [pltpu.VMEM((NC, EC), jnp.int32),
         pltpu.VMEM((NC, EC), jnp.int32),
         pltpu.VMEM((2, EC, 32), jnp.float32),
         pltpu.VMEM((HR, 16), jnp.float32),
         pltpu.VMEM((HR,), jnp.int32),
         pltpu.VMEM_SHARED((N, 32), jnp.float32),
         pltpu.VMEM_SHARED((NPAD, 32), jnp.float32),
         pltpu.VMEM_SHARED((HR, 16), jnp.float32)]
        + [pltpu.SemaphoreType.DMA] * 2))

_sc_layer2 = pl.kernel(
    _sc_layer2_body,
    out_type=[jax.ShapeDtypeStruct((2, NPAD, 16), jnp.float32)],
    mesh=_mesh,
    compiler_params=pltpu.CompilerParams(use_tc_tiling_on_sc=False),
    scratch_types=(
        [pltpu.VMEM((NC, EC), jnp.int32),
         pltpu.VMEM((NC, EC), jnp.int32),
         pltpu.VMEM((2, EC, 16), jnp.float32),
         pltpu.VMEM_SHARED((N, 16), jnp.float32),
         pltpu.VMEM_SHARED((NPAD, 16), jnp.float32)]
        + [pltpu.SemaphoreType.DMA] * 2))


def _tc_in_body(x_ref, wl_ref, wr_ref, bl_ref, p_ref, q_ref):
    x = x_ref[...]
    p_ref[...] = jnp.dot(x, wl_ref[...], preferred_element_type=jnp.float32)
    q_ref[...] = (jnp.dot(x, wr_ref[...], preferred_element_type=jnp.float32)
                  + bl_ref[...])


def _tc_mid_body(sum_ref, cnt_ref, q1_ref, wl_ref, wr_ref, bl_ref,
                 p2_ref, q2_ref):
    s = sum_ref[0, :N, :] + sum_ref[1, :N, :]
    c = cnt_ref[0, :N, :1] + cnt_ref[1, :N, :1]
    h = jnp.maximum(s / jnp.maximum(c, 1.0) + q1_ref[...], 0.0)
    p2_ref[...] = jnp.dot(h, wl_ref[...], preferred_element_type=jnp.float32)
    q2_ref[...] = (jnp.dot(h, wr_ref[...], preferred_element_type=jnp.float32)
                   + bl_ref[...])


def _tc_out_body(sum_ref, cnt_ref, q2_ref, wo_ref, bo_ref, o_ref):
    s = sum_ref[0, :N, :] + sum_ref[1, :N, :]
    c = cnt_ref[0, :N, :1] + cnt_ref[1, :N, :1]
    h = jnp.maximum(s / jnp.maximum(c, 1.0) + q2_ref[...], 0.0)
    o_ref[...] = jax.nn.sigmoid(
        jnp.dot(h, wo_ref[...], preferred_element_type=jnp.float32)
        + bo_ref[...])


_tc_out = pl.pallas_call(
    _tc_out_body,
    out_shape=jax.ShapeDtypeStruct((N, 1), jnp.float32))


_tc_in = pl.pallas_call(
    _tc_in_body,
    out_shape=[jax.ShapeDtypeStruct((N, 32), jnp.float32),
               jax.ShapeDtypeStruct((N, 32), jnp.float32)])
_tc_mid = pl.pallas_call(
    _tc_mid_body,
    out_shape=[jax.ShapeDtypeStruct((N, 16), jnp.float32),
               jax.ShapeDtypeStruct((N, 16), jnp.float32)])
def kernel(x, edge_index, Wl1, bl1, Wr1, Wl2, bl2, Wr2, Wo, bo):
    ei = edge_index.astype(jnp.int32)
    src = ei[0].reshape(NW, NC, EC)
    dst = ei[1].reshape(NW, NC, EC)
    z32 = jnp.zeros((RPT, 32), jnp.float32)
    z16 = jnp.zeros((RPT, 16), jnp.float32)
    z640 = jnp.zeros((HR, 16), jnp.float32)
    iota = jnp.arange(HR, dtype=jnp.int32)

    p1, q1 = _tc_in(x, Wl1, Wr1, bl1.reshape(1, 32))
    sum1, cnt = _sc_layer1(p1, src, dst, z32, z640, iota)
    cntp = cnt.reshape(2, HR * 16, 1)
    p2, q2 = _tc_mid(sum1, cntp, q1, Wl2, Wr2, bl2.reshape(1, 16))
    sum2, = _sc_layer2(p2, src, dst, z16)
    return _tc_out(sum2, cntp, q2, Wo, bo.reshape(1, 1))
